# Initial kernel scaffold; baseline (speedup 1.0000x reference)
import jax
import jax.numpy as jnp
from jax.experimental import pallas as pl


def kernel(x, edge_index, edge_attr, batch, We1, be1, W1a, b1a, W1b, b1b, We2, be2, W2a, b2a, W2b, b2b, Wfc, bfc):
    raise NotImplementedError("write your pallas kernel here")


# speedup vs baseline: 2.1836x; 2.1836x over previous
import functools

import jax
import jax.numpy as jnp
from jax import lax
from jax.experimental import pallas as pl
from jax.experimental.pallas import tpu as pltpu
from jax.experimental.pallas import tpu_sc as plsc

_HIGH = lax.Precision.HIGHEST

_NC = 2
_NS = 16
_L = 16

_G = 512


def _edge_emb_kernel(a_ref, We1_ref, be1_ref, We2_ref, be2_ref,
                     emb1_ref, emb2_ref):
    a = a_ref[...]
    emb1_ref[...] = lax.dot(a, We1_ref[...], precision=_HIGH) + be1_ref[...]
    emb2_ref[...] = lax.dot(a, We2_ref[...], precision=_HIGH) + be2_ref[...]


def _edge_embs(edge_attr, We1, be1, We2, be2):
    E, K = edge_attr.shape
    D1 = We1.shape[1]
    D2 = We2.shape[1]
    BE = 2000
    nb = E // BE
    return pl.pallas_call(
        _edge_emb_kernel,
        grid=(nb,),
        in_specs=[
            pl.BlockSpec((BE, K), lambda i: (i, 0)),
            pl.BlockSpec((K, D1), lambda i: (0, 0)),
            pl.BlockSpec((1, D1), lambda i: (0, 0)),
            pl.BlockSpec((K, D2), lambda i: (0, 0)),
            pl.BlockSpec((1, D2), lambda i: (0, 0)),
        ],
        out_specs=[
            pl.BlockSpec((BE, D1), lambda i: (i, 0)),
            pl.BlockSpec((BE, D2), lambda i: (i, 0)),
        ],
        out_shape=[
            jax.ShapeDtypeStruct((E, D1), jnp.float32),
            jax.ShapeDtypeStruct((E, D2), jnp.float32),
        ],
    )(edge_attr, We1, be1.reshape(1, D1), We2, be2.reshape(1, D2))


def _mlp_kernel(d_use, pad_to, x_ref, agg_ref, Wa_ref, ba_ref, Wb_ref, bb_ref,
                out_ref):
    h0 = (x_ref[...] + agg_ref[0] + agg_ref[1])[:, :d_use]
    t = jnp.maximum(lax.dot(h0, Wa_ref[...], precision=_HIGH) + ba_ref[...], 0.0)
    r = jnp.maximum(lax.dot(t, Wb_ref[...], precision=_HIGH) + bb_ref[...], 0.0)
    H = r.shape[1]
    if pad_to > H:
        r = jnp.concatenate(
            [r, jnp.zeros((r.shape[0], pad_to - H), jnp.float32)], axis=1)
    out_ref[...] = r


def _node_mlp(x, agg, Wa, ba, Wb, bb, d_use, pad_to):
    N, D = x.shape
    H = Wa.shape[1]
    R = 1000
    nb = N // R
    return pl.pallas_call(
        functools.partial(_mlp_kernel, d_use, pad_to),
        grid=(nb,),
        in_specs=[
            pl.BlockSpec((R, D), lambda i: (i, 0)),
            pl.BlockSpec((2, R, D), lambda i: (0, i, 0)),
            pl.BlockSpec((d_use, H), lambda i: (0, 0)),
            pl.BlockSpec((1, H), lambda i: (0, 0)),
            pl.BlockSpec((H, H), lambda i: (0, 0)),
            pl.BlockSpec((1, H), lambda i: (0, 0)),
        ],
        out_specs=pl.BlockSpec((R, pad_to), lambda i: (i, 0)),
        out_shape=jax.ShapeDtypeStruct((N, pad_to), jnp.float32),
    )(x, agg, Wa, ba.reshape(1, H), Wb, bb.reshape(1, H))


def _pool_fc_kernel(nb, batch_ref, h_ref, Wfc_ref, bfc_ref, out_ref, acc_ref):
    i = pl.program_id(0)

    @pl.when(i == 0)
    def _():
        acc_ref[...] = jnp.zeros_like(acc_ref)

    b = batch_ref[0]
    R = b.shape[1]
    onehot = (lax.broadcasted_iota(jnp.int32, (_G, R), 0) == b
              ).astype(jnp.float32)
    acc_ref[...] += lax.dot(onehot, h_ref[...], precision=_HIGH)

    @pl.when(i == nb - 1)
    def _():
        out_ref[...] = (lax.dot(acc_ref[...], Wfc_ref[...], precision=_HIGH)
                        + bfc_ref[...])


def _pool_fc(h, batch, Wfc, bfc):
    N, H = h.shape
    OUT = Wfc.shape[1]
    R = 1000
    nb = N // R
    batch3 = batch.reshape(nb, 1, R)
    return pl.pallas_call(
        functools.partial(_pool_fc_kernel, nb),
        grid=(nb,),
        in_specs=[
            pl.BlockSpec((1, 1, R), lambda i: (i, 0, 0)),
            pl.BlockSpec((R, H), lambda i: (i, 0)),
            pl.BlockSpec((H, OUT), lambda i: (0, 0)),
            pl.BlockSpec((1, OUT), lambda i: (0, 0)),
        ],
        out_specs=pl.BlockSpec((_G, OUT), lambda i: (0, 0)),
        out_shape=jax.ShapeDtypeStruct((_G, OUT), jnp.float32),
        scratch_shapes=[pltpu.VMEM((_G, H), jnp.float32)],
    )(batch3, h, Wfc, bfc.reshape(1, OUT))


def _make_segsum(N, E, D, D_emb):
    B = 80
    e_per_tile = E // (_NC * _NS)
    nblocks = e_per_tile // B
    rows_main = (N // (8 * _NS)) * 8
    tail = N - _NS * rows_main
    ZR = 104
    while rows_main % ZR:
        ZR -= 8
    nz = rows_main // ZR
    mesh = plsc.VectorSubcoreMesh(core_axis_name="c", subcore_axis_name="s")

    @functools.partial(
        pl.kernel,
        out_type=jax.ShapeDtypeStruct((_NC, N, D), jnp.float32),
        mesh=mesh,
        scratch_types=[
            pltpu.VMEM((B,), jnp.int32),
            pltpu.VMEM((B,), jnp.int32),
            pltpu.VMEM((B, D), jnp.float32),
            pltpu.VMEM((B, D_emb), jnp.float32),
            pltpu.VMEM((ZR, D), jnp.float32),
            pltpu.VMEM_SHARED((N, D), jnp.float32),
        ],
    )
    def segsum(x_hbm, emb_hbm, src_hbm, dst_hbm, out_hbm,
               src_v, dst_v, xg_v, emb_v, zero_v, acc_sh):
        cid = lax.axis_index("c")
        sid = lax.axis_index("s")

        @pl.loop(0, ZR)
        def _(r):
            for c in range(D // _L):
                zero_v[r, pl.ds(c * _L, _L)] = jnp.zeros((_L,), jnp.float32)

        for j in range(nz):
            pltpu.sync_copy(
                zero_v, acc_sh.at[pl.ds(sid * rows_main + j * ZR, ZR)])
        if tail:
            @pl.when(sid == _NS - 1)
            def _():
                pltpu.sync_copy(zero_v.at[pl.ds(0, tail)],
                                acc_sh.at[pl.ds(_NS * rows_main, tail)])
        plsc.subcore_barrier()

        base = (cid * _NS + sid) * e_per_tile

        @pl.loop(0, nblocks)
        def _(i):
            off = base + i * B
            pltpu.sync_copy(src_hbm.at[pl.ds(off, B)], src_v)
            pltpu.sync_copy(dst_hbm.at[pl.ds(off, B)], dst_v)
            pltpu.sync_copy(x_hbm.at[src_v], xg_v)
            pltpu.sync_copy(emb_hbm.at[pl.ds(off, B)], emb_v)

            @pl.loop(0, B)
            def _(r):
                for c in range(D_emb // _L):
                    sl = pl.ds(c * _L, _L)
                    xg_v[r, sl] = jnp.maximum(xg_v[r, sl] + emb_v[r, sl], 0.0)

            pltpu.sync_copy(xg_v, acc_sh.at[dst_v], add=True)

        plsc.subcore_barrier()
        pltpu.sync_copy(
            acc_sh.at[pl.ds(sid * rows_main, rows_main)],
            out_hbm.at[cid, pl.ds(sid * rows_main, rows_main)])
        if tail:
            @pl.when(sid == _NS - 1)
            def _():
                pltpu.sync_copy(
                    acc_sh.at[pl.ds(_NS * rows_main, tail)],
                    out_hbm.at[cid, pl.ds(_NS * rows_main, tail)])

    return segsum


def kernel(x, edge_index, edge_attr, batch, We1, be1, W1a, b1a, W1b, b1b,
           We2, be2, W2a, b2a, W2b, b2b, Wfc, bfc):
    N, D1 = x.shape
    E = edge_attr.shape[0]
    H = W1a.shape[1]
    src = edge_index[0]
    dst = edge_index[1]

    emb1, emb2 = _edge_embs(edge_attr, We1, be1, We2, be2)

    agg1 = _make_segsum(N, E, D1, D1)(x, emb1, src, dst)
    h = _node_mlp(x, agg1, W1a, b1a, W1b, b1b, d_use=D1, pad_to=D1)

    agg2 = _make_segsum(N, E, D1, H)(h, emb2, src, dst)
    h2 = _node_mlp(h, agg2, W2a, b2a, W2b, b2b, d_use=H, pad_to=H)

    return _pool_fc(h2, batch, Wfc, bfc)


# speedup vs baseline: 3.7073x; 1.6978x over previous
import functools

import jax
import jax.numpy as jnp
from jax import lax
from jax.experimental import pallas as pl
from jax.experimental.pallas import tpu as pltpu
from jax.experimental.pallas import tpu_sc as plsc

_HIGH = lax.Precision.HIGHEST

_NC = 2
_NS = 16
_L = 16

_G = 512
_B = 80


def _chunk_geom(E):
    nblocks = E // (_NC * _NS * _B)
    ch = 25
    while nblocks % ch:
        ch -= 1
    return nblocks, ch, nblocks // ch


def _edge_emb_kernel(a_ref, We1_ref, be1_ref, We2_ref, be2_ref,
                     emb1_ref, emb2_ref):
    a = a_ref[...]
    emb1_ref[...] = lax.dot(a, We1_ref[...], precision=_HIGH) + be1_ref[...]
    emb2_ref[...] = lax.dot(a, We2_ref[...], precision=_HIGH) + be2_ref[...]


def _edge_embs(edge_attr, We1, be1, We2, be2):
    E, K = edge_attr.shape
    D1 = We1.shape[1]
    D2 = We2.shape[1]
    BE = 2000
    nb = E // BE
    return pl.pallas_call(
        _edge_emb_kernel,
        grid=(nb,),
        in_specs=[
            pl.BlockSpec((BE, K), lambda i: (i, 0)),
            pl.BlockSpec((K, D1), lambda i: (0, 0)),
            pl.BlockSpec((1, D1), lambda i: (0, 0)),
            pl.BlockSpec((K, D2), lambda i: (0, 0)),
            pl.BlockSpec((1, D2), lambda i: (0, 0)),
        ],
        out_specs=[
            pl.BlockSpec((BE, D1), lambda i: (i, 0)),
            pl.BlockSpec((BE, D2), lambda i: (i, 0)),
        ],
        out_shape=[
            jax.ShapeDtypeStruct((E, D1), jnp.float32),
            jax.ShapeDtypeStruct((E, D2), jnp.float32),
        ],
    )(edge_attr, We1, be1.reshape(1, D1), We2, be2.reshape(1, D2))


def _mlp_kernel(d_use, pad_to, x_ref, agg_ref, Wa_ref, ba_ref, Wb_ref, bb_ref,
                out_ref):
    h0 = (x_ref[...] + agg_ref[0] + agg_ref[1])[:, :d_use]
    t = jnp.maximum(lax.dot(h0, Wa_ref[...], precision=_HIGH) + ba_ref[...], 0.0)
    r = jnp.maximum(lax.dot(t, Wb_ref[...], precision=_HIGH) + bb_ref[...], 0.0)
    H = r.shape[1]
    if pad_to > H:
        r = jnp.concatenate(
            [r, jnp.zeros((r.shape[0], pad_to - H), jnp.float32)], axis=1)
    out_ref[...] = r


def _node_mlp(x, agg, Wa, ba, Wb, bb, d_use, pad_to):
    N, D = x.shape
    H = Wa.shape[1]
    R = 1000
    nb = N // R
    return pl.pallas_call(
        functools.partial(_mlp_kernel, d_use, pad_to),
        grid=(nb,),
        in_specs=[
            pl.BlockSpec((R, D), lambda i: (i, 0)),
            pl.BlockSpec((2, R, D), lambda i: (0, i, 0)),
            pl.BlockSpec((d_use, H), lambda i: (0, 0)),
            pl.BlockSpec((1, H), lambda i: (0, 0)),
            pl.BlockSpec((H, H), lambda i: (0, 0)),
            pl.BlockSpec((1, H), lambda i: (0, 0)),
        ],
        out_specs=pl.BlockSpec((R, pad_to), lambda i: (i, 0)),
        out_shape=jax.ShapeDtypeStruct((N, pad_to), jnp.float32),
    )(x, agg, Wa, ba.reshape(1, H), Wb, bb.reshape(1, H))


def _pool_fc_kernel(nb, batch_ref, h_ref, Wfc_ref, bfc_ref, out_ref, acc_ref):
    i = pl.program_id(0)

    @pl.when(i == 0)
    def _():
        acc_ref[...] = jnp.zeros_like(acc_ref)

    b = batch_ref[0]
    R = b.shape[1]
    onehot = (lax.broadcasted_iota(jnp.int32, (_G, R), 0) == b
              ).astype(jnp.float32)
    acc_ref[...] += lax.dot(onehot, h_ref[...], precision=_HIGH)

    @pl.when(i == nb - 1)
    def _():
        out_ref[...] = (lax.dot(acc_ref[...], Wfc_ref[...], precision=_HIGH)
                        + bfc_ref[...])


def _pool_fc(h, batch, Wfc, bfc):
    N, H = h.shape
    OUT = Wfc.shape[1]
    R = 1000
    nb = N // R
    batch3 = batch.reshape(nb, 1, R)
    return pl.pallas_call(
        functools.partial(_pool_fc_kernel, nb),
        grid=(nb,),
        in_specs=[
            pl.BlockSpec((1, 1, R), lambda i: (i, 0, 0)),
            pl.BlockSpec((R, H), lambda i: (i, 0)),
            pl.BlockSpec((H, OUT), lambda i: (0, 0)),
            pl.BlockSpec((1, OUT), lambda i: (0, 0)),
        ],
        out_specs=pl.BlockSpec((_G, OUT), lambda i: (0, 0)),
        out_shape=jax.ShapeDtypeStruct((_G, OUT), jnp.float32),
        scratch_shapes=[pltpu.VMEM((_G, H), jnp.float32)],
    )(batch3, h, Wfc, bfc.reshape(1, OUT))


def _make_segsum(N, E, D, D_emb):
    B = _B
    e_per_tile = E // (_NC * _NS)
    nblocks, CH, nch = _chunk_geom(E)
    NW = _NC * _NS
    rows_main = (N // (8 * _NS)) * 8
    tail = N - _NS * rows_main
    ZR = 104
    while rows_main % ZR:
        ZR -= 8
    nz = rows_main // ZR
    mesh = plsc.VectorSubcoreMesh(core_axis_name="c", subcore_axis_name="s")

    @functools.partial(
        pl.kernel,
        out_type=jax.ShapeDtypeStruct((_NC, N, D), jnp.float32),
        mesh=mesh,
        scratch_types=[
            pltpu.VMEM((CH, B), jnp.int32),
            pltpu.VMEM((CH, B), jnp.int32),
            pltpu.VMEM((B, D), jnp.float32),
            pltpu.VMEM((B, D), jnp.float32),
            pltpu.VMEM((B, D_emb), jnp.float32),
            pltpu.VMEM((B, D_emb), jnp.float32),
            pltpu.VMEM_SHARED((N, D), jnp.float32),
            pltpu.SemaphoreType.DMA,
            pltpu.SemaphoreType.DMA,
            pltpu.SemaphoreType.DMA,
            pltpu.SemaphoreType.DMA,
        ],
    )
    def segsum(x_hbm, emb_hbm, src_hbm, dst_hbm, out_hbm,
               src_v, dst_v, xg0, xg1, em0, em1, acc_sh,
               g0, g1, e0, e1):
        cid = lax.axis_index("c")
        sid = lax.axis_index("s")
        wid = cid * _NS + sid
        xg = (xg0, xg1)
        em = (em0, em1)
        gsem = (g0, g1)
        esem = (e0, e1)

        @pl.loop(0, B)
        def _(r):
            for c in range(D // _L):
                xg0[r, pl.ds(c * _L, _L)] = jnp.zeros((_L,), jnp.float32)

        nz, rem = rows_main // B, rows_main % B
        for j in range(nz):
            pltpu.sync_copy(
                xg0, acc_sh.at[pl.ds(sid * rows_main + j * B, B)])
        if rem:
            pltpu.sync_copy(xg0.at[pl.ds(0, rem)],
                            acc_sh.at[pl.ds(sid * rows_main + nz * B, rem)])
        if tail:
            @pl.when(sid == _NS - 1)
            def _():
                pltpu.sync_copy(xg0.at[pl.ds(0, tail)],
                                acc_sh.at[pl.ds(_NS * rows_main, tail)])
        plsc.subcore_barrier()

        base_e = wid * e_per_tile

        def issue(k, j, b):
            pltpu.async_copy(x_hbm.at[src_v.at[j]], xg[b], gsem[b])
            pltpu.async_copy(
                emb_hbm.at[pl.ds(base_e + (k * CH + j) * B, B)],
                em[b], esem[b])

        def wait_in(k, j, b):
            pltpu.make_async_copy(x_hbm.at[src_v.at[j]], xg[b], gsem[b]).wait()
            pltpu.make_async_copy(
                emb_hbm.at[pl.ds(base_e + (k * CH + j) * B, B)],
                em[b], esem[b]).wait()

        def compute(b):
            xv, ev = xg[b], em[b]

            @pl.loop(0, B)
            def _(r):
                for c in range(D_emb // _L):
                    sl = pl.ds(c * _L, _L)
                    xv[r, sl] = jnp.maximum(xv[r, sl] + ev[r, sl], 0.0)

        def step(k, j, b, lookahead):
            if lookahead:
                issue(k, j + 1, 1 - b)
            wait_in(k, j, b)
            compute(b)
            pltpu.sync_copy(xg[b], acc_sh.at[dst_v.at[j]], add=True)

        @pl.loop(0, nch)
        def _(k):
            pltpu.sync_copy(src_hbm.at[wid, k], src_v)
            pltpu.sync_copy(dst_hbm.at[wid, k], dst_v)
            issue(k, 0, 0)

            npairs = CH // 2 if CH % 2 else CH // 2 - 1

            @pl.loop(0, npairs)
            def _(p):
                step(k, 2 * p, 0, lookahead=True)
                step(k, 2 * p + 1, 1, lookahead=True)
            if CH % 2:
                step(k, CH - 1, 0, lookahead=False)
            else:
                step(k, CH - 2, 0, lookahead=True)
                step(k, CH - 1, 1, lookahead=False)

        plsc.subcore_barrier()
        pltpu.sync_copy(
            acc_sh.at[pl.ds(sid * rows_main, rows_main)],
            out_hbm.at[cid, pl.ds(sid * rows_main, rows_main)])
        if tail:
            @pl.when(sid == _NS - 1)
            def _():
                pltpu.sync_copy(
                    acc_sh.at[pl.ds(_NS * rows_main, tail)],
                    out_hbm.at[cid, pl.ds(_NS * rows_main, tail)])

    return segsum


def kernel(x, edge_index, edge_attr, batch, We1, be1, W1a, b1a, W1b, b1b,
           We2, be2, W2a, b2a, W2b, b2b, Wfc, bfc):
    N, D1 = x.shape
    E = edge_attr.shape[0]
    H = W1a.shape[1]
    NW = _NC * _NS
    nblocks, CH, nch = _chunk_geom(E)
    src = edge_index[0].reshape(NW, nch, CH, _B)
    dst = edge_index[1].reshape(NW, nch, CH, _B)

    emb1, emb2 = _edge_embs(edge_attr, We1, be1, We2, be2)

    agg1 = _make_segsum(N, E, D1, D1)(x, emb1, src, dst)
    h = _node_mlp(x, agg1, W1a, b1a, W1b, b1b, d_use=D1, pad_to=D1)

    agg2 = _make_segsum(N, E, D1, H)(h, emb2, src, dst)
    h2 = _node_mlp(h, agg2, W2a, b2a, W2b, b2b, d_use=H, pad_to=H)

    return _pool_fc(h2, batch, Wfc, bfc)


# speedup vs baseline: 4.9664x; 1.3396x over previous
import functools

import jax
import jax.numpy as jnp
from jax import lax
from jax.experimental import pallas as pl
from jax.experimental.pallas import tpu as pltpu
from jax.experimental.pallas import tpu_sc as plsc

_HIGH = lax.Precision.HIGHEST

_NC = 2
_NS = 16
_L = 16

_G = 512
_B = 80


def _chunk_geom(E):
    nblocks = E // (_NC * _NS * _B)
    ch = 25
    while nblocks % ch:
        ch -= 1
    return nblocks, ch, nblocks // ch


def _edge_emb_kernel(at_ref, We_ref, be_ref, emb_ref):
    emb_ref[...] = lax.dot_general(
        at_ref[...], We_ref[...], (((0,), (0,)), ((), ())),
        precision=_HIGH) + be_ref[...]


def _edge_emb(edge_attr_t, We, be):
    K, E = edge_attr_t.shape
    D = We.shape[1]
    BE = 6400
    nb = E // BE
    return pl.pallas_call(
        _edge_emb_kernel,
        grid=(nb,),
        in_specs=[
            pl.BlockSpec((K, BE), lambda i: (0, i)),
            pl.BlockSpec((K, D), lambda i: (0, 0)),
            pl.BlockSpec((1, D), lambda i: (0, 0)),
        ],
        out_specs=pl.BlockSpec((BE, D), lambda i: (i, 0)),
        out_shape=jax.ShapeDtypeStruct((E, D), jnp.float32),
    )(edge_attr_t, We, be.reshape(1, D))


def _mlp_kernel(d_use, pad_to, x_ref, agg_ref, Wa_ref, ba_ref, Wb_ref, bb_ref,
                out_ref):
    h0 = x_ref[...][:, :d_use] + (agg_ref[0] + agg_ref[1])[:, :d_use]
    t = jnp.maximum(lax.dot(h0, Wa_ref[...], precision=_HIGH) + ba_ref[...], 0.0)
    r = jnp.maximum(lax.dot(t, Wb_ref[...], precision=_HIGH) + bb_ref[...], 0.0)
    H = r.shape[1]
    if pad_to > H:
        r = jnp.concatenate(
            [r, jnp.zeros((r.shape[0], pad_to - H), jnp.float32)], axis=1)
    out_ref[...] = r


def _node_mlp(x, agg, Wa, ba, Wb, bb, d_use, pad_to):
    N, D = x.shape
    D_agg = agg.shape[2]
    H = Wa.shape[1]
    R = 1000
    nb = N // R
    return pl.pallas_call(
        functools.partial(_mlp_kernel, d_use, pad_to),
        grid=(nb,),
        in_specs=[
            pl.BlockSpec((R, D), lambda i: (i, 0)),
            pl.BlockSpec((2, R, D_agg), lambda i: (0, i, 0)),
            pl.BlockSpec((d_use, H), lambda i: (0, 0)),
            pl.BlockSpec((1, H), lambda i: (0, 0)),
            pl.BlockSpec((H, H), lambda i: (0, 0)),
            pl.BlockSpec((1, H), lambda i: (0, 0)),
        ],
        out_specs=pl.BlockSpec((R, pad_to), lambda i: (i, 0)),
        out_shape=jax.ShapeDtypeStruct((N, pad_to), jnp.float32),
    )(x, agg, Wa, ba.reshape(1, H), Wb, bb.reshape(1, H))


def _pool_fc_kernel(nb, batch_ref, h_ref, Wfc_ref, bfc_ref, out_ref, acc_ref):
    i = pl.program_id(0)

    @pl.when(i == 0)
    def _():
        acc_ref[...] = jnp.zeros_like(acc_ref)

    b = batch_ref[0]
    R = b.shape[1]
    onehot = (lax.broadcasted_iota(jnp.int32, (_G, R), 0) == b
              ).astype(jnp.float32)
    acc_ref[...] += lax.dot(onehot, h_ref[...], precision=_HIGH)

    @pl.when(i == nb - 1)
    def _():
        out_ref[...] = (lax.dot(acc_ref[...], Wfc_ref[...], precision=_HIGH)
                        + bfc_ref[...])


def _pool_fc(h, batch, Wfc, bfc):
    N, H = h.shape
    OUT = Wfc.shape[1]
    R = 1000
    nb = N // R
    batch3 = batch.reshape(nb, 1, R)
    return pl.pallas_call(
        functools.partial(_pool_fc_kernel, nb),
        grid=(nb,),
        in_specs=[
            pl.BlockSpec((1, 1, R), lambda i: (i, 0, 0)),
            pl.BlockSpec((R, H), lambda i: (i, 0)),
            pl.BlockSpec((H, OUT), lambda i: (0, 0)),
            pl.BlockSpec((1, OUT), lambda i: (0, 0)),
        ],
        out_specs=pl.BlockSpec((_G, OUT), lambda i: (0, 0)),
        out_shape=jax.ShapeDtypeStruct((_G, OUT), jnp.float32),
        scratch_shapes=[pltpu.VMEM((_G, H), jnp.float32)],
    )(batch3, h, Wfc, bfc.reshape(1, OUT))


def _make_segsum(N, E, D, D_emb):
    B = _B
    e_per_tile = E // (_NC * _NS)
    nblocks, CH, nch = _chunk_geom(E)
    NW = _NC * _NS
    rows_main = (N // (8 * _NS)) * 8
    tail = N - _NS * rows_main
    ZR = 104
    while rows_main % ZR:
        ZR -= 8
    nz = rows_main // ZR
    mesh = plsc.VectorSubcoreMesh(core_axis_name="c", subcore_axis_name="s")

    narrow = False
    A = D_emb if narrow else D
    scratch = [
        pltpu.VMEM((CH, B), jnp.int32),
        pltpu.VMEM((CH, B), jnp.int32),
        pltpu.VMEM((B, D), jnp.float32),
        pltpu.VMEM((B, D), jnp.float32),
        pltpu.VMEM((B, D_emb), jnp.float32),
        pltpu.VMEM((B, D_emb), jnp.float32),
    ]
    if narrow:
        scratch.append(pltpu.VMEM((B, D_emb), jnp.float32))
    scratch += [
        pltpu.VMEM_SHARED((N, A), jnp.float32),
        pltpu.SemaphoreType.DMA,
        pltpu.SemaphoreType.DMA,
        pltpu.SemaphoreType.DMA,
        pltpu.SemaphoreType.DMA,
    ]

    @functools.partial(
        pl.kernel,
        out_type=jax.ShapeDtypeStruct((_NC, N, A), jnp.float32),
        mesh=mesh,
        scratch_types=scratch,
    )
    def segsum(x_hbm, emb_hbm, src_hbm, dst_hbm, out_hbm,
               src_v, dst_v, xg0, xg1, em0, em1, *rest):
        if narrow:
            msg, acc_sh, g0, g1, e0, e1 = rest
        else:
            acc_sh, g0, g1, e0, e1 = rest
        cid = lax.axis_index("c")
        sid = lax.axis_index("s")
        wid = cid * _NS + sid
        xg = (xg0, xg1)
        em = (em0, em1)
        gsem = (g0, g1)
        esem = (e0, e1)
        zsrc = msg if narrow else xg0

        @pl.loop(0, B)
        def _(r):
            for c in range(A // _L):
                zsrc[r, pl.ds(c * _L, _L)] = jnp.zeros((_L,), jnp.float32)

        nz, rem = rows_main // B, rows_main % B
        for j in range(nz):
            pltpu.sync_copy(
                zsrc, acc_sh.at[pl.ds(sid * rows_main + j * B, B)])
        if rem:
            pltpu.sync_copy(zsrc.at[pl.ds(0, rem)],
                            acc_sh.at[pl.ds(sid * rows_main + nz * B, rem)])
        if tail:
            @pl.when(sid == _NS - 1)
            def _():
                pltpu.sync_copy(zsrc.at[pl.ds(0, tail)],
                                acc_sh.at[pl.ds(_NS * rows_main, tail)])
        plsc.subcore_barrier()

        base_e = wid * e_per_tile

        def issue(k, j, b):
            pltpu.async_copy(x_hbm.at[src_v.at[j]], xg[b], gsem[b])
            pltpu.async_copy(
                emb_hbm.at[pl.ds(base_e + (k * CH + j) * B, B)],
                em[b], esem[b])

        def wait_in(k, j, b):
            pltpu.make_async_copy(x_hbm.at[src_v.at[j]], xg[b], gsem[b]).wait()
            pltpu.make_async_copy(
                emb_hbm.at[pl.ds(base_e + (k * CH + j) * B, B)],
                em[b], esem[b]).wait()

        def compute(b):
            xv, ev = xg[b], em[b]
            dst_buf = msg if narrow else xv

            @pl.loop(0, B)
            def _(r):
                for c in range(D_emb // _L):
                    sl = pl.ds(c * _L, _L)
                    dst_buf[r, sl] = jnp.maximum(xv[r, sl] + ev[r, sl], 0.0)

        def step(k, j, b, lookahead):
            if lookahead:
                issue(k, j + 1, 1 - b)
            wait_in(k, j, b)
            compute(b)
            pltpu.sync_copy(msg if narrow else xg[b],
                            acc_sh.at[dst_v.at[j]], add=True)

        @pl.loop(0, nch)
        def _(k):
            pltpu.sync_copy(src_hbm.at[wid, k], src_v)
            pltpu.sync_copy(dst_hbm.at[wid, k], dst_v)
            issue(k, 0, 0)

            npairs = CH // 2 if CH % 2 else CH // 2 - 1

            @pl.loop(0, npairs)
            def _(p):
                step(k, 2 * p, 0, lookahead=True)
                step(k, 2 * p + 1, 1, lookahead=True)
            if CH % 2:
                step(k, CH - 1, 0, lookahead=False)
            else:
                step(k, CH - 2, 0, lookahead=True)
                step(k, CH - 1, 1, lookahead=False)

        plsc.subcore_barrier()
        pltpu.sync_copy(
            acc_sh.at[pl.ds(sid * rows_main, rows_main)],
            out_hbm.at[cid, pl.ds(sid * rows_main, rows_main)])
        if tail:
            @pl.when(sid == _NS - 1)
            def _():
                pltpu.sync_copy(
                    acc_sh.at[pl.ds(_NS * rows_main, tail)],
                    out_hbm.at[cid, pl.ds(_NS * rows_main, tail)])

    return segsum


def kernel(x, edge_index, edge_attr, batch, We1, be1, W1a, b1a, W1b, b1b,
           We2, be2, W2a, b2a, W2b, b2b, Wfc, bfc):
    N, D1 = x.shape
    E = edge_attr.shape[0]
    H = W1a.shape[1]
    NW = _NC * _NS
    nblocks, CH, nch = _chunk_geom(E)
    src = edge_index[0].reshape(NW, nch, CH, _B)
    dst = edge_index[1].reshape(NW, nch, CH, _B)

    at = edge_attr.T

    emb1 = _edge_emb(at, We1, be1)
    agg1 = _make_segsum(N, E, D1, D1)(x, emb1, src, dst)
    emb2 = _edge_emb(at, We2, be2)
    h = _node_mlp(x, agg1, W1a, b1a, W1b, b1b, d_use=D1, pad_to=D1)

    agg2 = _make_segsum(N, E, D1, H)(h, emb2, src, dst)
    h2 = _node_mlp(h, agg2, W2a, b2a, W2b, b2b, d_use=H, pad_to=H)

    return _pool_fc(h2, batch, Wfc, bfc)
